# baseline (speedup 1.0000x reference)
import functools

import jax
import jax.numpy as jnp
from jax.experimental import pallas as pl
from jax.experimental.pallas import tpu as pltpu

B, DIM, H, W = 8, 192, 14, 14
D_TEXT = 512
E = 8
K = 2
HP, WP = H + 2, W + 2


def _gate_kernel(xp_ref, tf_ref, wgi_ref, wgt_ref, bg_ref,
                 ti_ref, tv_ref, aux_ref):
    xp = xp_ref[:]
    pooled = jnp.sum(xp, axis=(1, 2)) * (1.0 / (H * W))
    logits = (jnp.dot(pooled, wgi_ref[:], preferred_element_type=jnp.float32)
              + jnp.dot(tf_ref[:], wgt_ref[:], preferred_element_type=jnp.float32)
              + bg_ref[:])
    z = logits - jnp.max(logits, axis=-1, keepdims=True)
    ez = jnp.exp(z)
    probs = ez / jnp.sum(ez, axis=-1, keepdims=True)

    lane = jax.lax.broadcasted_iota(jnp.int32, (B, E), 1)
    v1 = jnp.max(probs, axis=-1, keepdims=True)
    i1 = jnp.min(jnp.where(probs == v1, lane, E), axis=-1, keepdims=True)
    p2 = jnp.where(lane == i1, -1.0, probs)
    v2 = jnp.max(p2, axis=-1, keepdims=True)
    i2 = jnp.min(jnp.where(p2 == v2, lane, E), axis=-1, keepdims=True)

    denom = 1.0 / (v1 + v2 + 1e-9)
    v1n = v1 * denom
    v2n = v2 * denom

    ti_ref[:] = jnp.concatenate([i1, i2], axis=1)
    tv_ref[:] = jnp.concatenate([v1n, v2n], axis=1)

    one1 = jnp.where(lane == i1, v1n, 0.0)
    one2 = jnp.where(lane == i2, v2n, 0.0)
    imp = jnp.sum(one1 + one2, axis=0, keepdims=True)
    m = jnp.mean(imp, axis=-1, keepdims=True)
    var = jnp.mean((imp - m) ** 2, axis=-1, keepdims=True)
    aux_ref[:] = var / (m * m + 1e-10)


def _col_shift(x3, s):
    if s == 0:
        return x3
    zrow = jnp.zeros((HP, 1, DIM), jnp.float32)
    if s == -1:
        return jnp.concatenate([zrow, x3[:, :WP - 1]], axis=1)
    return jnp.concatenate([x3[:, 1:], zrow], axis=1)


def _row_shift(x3, s):
    if s == 0:
        return x3
    zpl = jnp.zeros((1, WP, DIM), jnp.float32)
    if s == -1:
        return jnp.concatenate([zpl, x3[:HP - 1]], axis=0)
    return jnp.concatenate([x3[1:], zpl], axis=0)


def _conv3x3_padded(x3, w_ref):
    cols = [_col_shift(x3, s) for s in (-1, 0, 1)]
    acc = None
    for t in range(9):
        dy, dx = t // 3, t % 3
        xs = _row_shift(cols[dx], dy - 1).reshape(HP * WP, DIM)
        z = jnp.dot(xs, w_ref[0, t], preferred_element_type=jnp.float32)
        acc = z if acc is None else acc + z
    return acc.reshape(HP, WP, DIM)


def _expert_kernel(ti_ref, gv_ref, xp_ref, w1_ref, b1_ref, w2_ref, b2_ref,
                   out_ref):
    b = pl.program_id(0)
    k = pl.program_id(1)

    ii = jax.lax.broadcasted_iota(jnp.int32, (HP, WP, 1), 0)
    jj = jax.lax.broadcasted_iota(jnp.int32, (HP, WP, 1), 1)
    interior = (ii >= 1) & (ii <= H) & (jj >= 1) & (jj <= W)

    x3 = xp_ref[0]
    h = _conv3x3_padded(x3, w1_ref) + b1_ref[0, 0][None, None, :]
    h = 0.5 * h * (1.0 + jax.lax.erf(h * 0.7071067811865476))
    h = jnp.where(interior, h, 0.0)

    y = _conv3x3_padded(h, w2_ref) + b2_ref[0, 0][None, None, :]
    contrib = y * gv_ref[b, k]

    @pl.when(k == 0)
    def _():
        out_ref[0] = contrib

    @pl.when(k != 0)
    def _():
        out_ref[0] = out_ref[0] + contrib


@jax.jit
def kernel(x, text_feature, Wg, bg, w1, b1, w2, b2):
    xp = jnp.pad(x.transpose(0, 2, 3, 1),
                 ((0, 0), (1, 1), (1, 1), (0, 0)))
    w1r = w1.transpose(0, 3, 4, 2, 1).reshape(E, 9, DIM, DIM)
    w2r = w2.transpose(0, 3, 4, 2, 1).reshape(E, 9, DIM, DIM)
    b1r = b1.reshape(E, 1, DIM)
    b2r = b2.reshape(E, 1, DIM)

    ti, tv, aux = pl.pallas_call(
        _gate_kernel,
        out_shape=(
            jax.ShapeDtypeStruct((B, K), jnp.int32),
            jax.ShapeDtypeStruct((B, K), jnp.float32),
            jax.ShapeDtypeStruct((1, 1), jnp.float32),
        ),
    )(xp, text_feature, Wg[:DIM], Wg[DIM:], bg.reshape(1, E))

    grid_spec = pltpu.PrefetchScalarGridSpec(
        num_scalar_prefetch=2,
        grid=(B, K),
        in_specs=[
            pl.BlockSpec((1, HP, WP, DIM), lambda b, k, ti, tv: (b, 0, 0, 0)),
            pl.BlockSpec((1, 9, DIM, DIM),
                         lambda b, k, ti, tv: (ti[b, k], 0, 0, 0)),
            pl.BlockSpec((1, 1, DIM), lambda b, k, ti, tv: (ti[b, k], 0, 0)),
            pl.BlockSpec((1, 9, DIM, DIM),
                         lambda b, k, ti, tv: (ti[b, k], 0, 0, 0)),
            pl.BlockSpec((1, 1, DIM), lambda b, k, ti, tv: (ti[b, k], 0, 0)),
        ],
        out_specs=pl.BlockSpec((1, HP, WP, DIM),
                               lambda b, k, ti, tv: (b, 0, 0, 0)),
    )
    out = pl.pallas_call(
        _expert_kernel,
        grid_spec=grid_spec,
        out_shape=jax.ShapeDtypeStruct((B, HP, WP, DIM), jnp.float32),
    )(ti, tv, xp, w1r, b1r, w2r, b2r)

    return out[:, 1:1 + H, 1:1 + W, :].transpose(0, 3, 1, 2), aux[0, 0]


# speedup vs baseline: 2.1000x; 2.1000x over previous
import functools

import jax
import jax.numpy as jnp
from jax.experimental import pallas as pl
from jax.experimental.pallas import tpu as pltpu

B, DIM, H, W = 8, 192, 14, 14
D_TEXT = 512
E = 8
K = 2
HP, WP = H + 2, W + 2


def _gate_kernel(xp_ref, tf_ref, wgi_ref, wgt_ref, bg_ref,
                 ti_ref, tv_ref, aux_ref):
    xp = xp_ref[:]
    pooled = jnp.sum(xp, axis=(1, 2)) * (1.0 / (H * W))
    logits = (jnp.dot(pooled, wgi_ref[:], preferred_element_type=jnp.float32)
              + jnp.dot(tf_ref[:], wgt_ref[:], preferred_element_type=jnp.float32)
              + bg_ref[:])
    z = logits - jnp.max(logits, axis=-1, keepdims=True)
    ez = jnp.exp(z)
    probs = ez / jnp.sum(ez, axis=-1, keepdims=True)

    lane = jax.lax.broadcasted_iota(jnp.int32, (B, E), 1)
    v1 = jnp.max(probs, axis=-1, keepdims=True)
    i1 = jnp.min(jnp.where(probs == v1, lane, E), axis=-1, keepdims=True)
    p2 = jnp.where(lane == i1, -1.0, probs)
    v2 = jnp.max(p2, axis=-1, keepdims=True)
    i2 = jnp.min(jnp.where(p2 == v2, lane, E), axis=-1, keepdims=True)

    denom = 1.0 / (v1 + v2 + 1e-9)
    v1n = v1 * denom
    v2n = v2 * denom

    ti_ref[:] = jnp.concatenate([i1, i2], axis=1)
    tv_ref[:] = jnp.concatenate([v1n, v2n], axis=1)

    one1 = jnp.where(lane == i1, v1n, 0.0)
    one2 = jnp.where(lane == i2, v2n, 0.0)
    imp = jnp.sum(one1 + one2, axis=0, keepdims=True)
    m = jnp.mean(imp, axis=-1, keepdims=True)
    var = jnp.mean((imp - m) ** 2, axis=-1, keepdims=True)
    aux_ref[:] = var / (m * m + 1e-10)


def _col_shift(x3, s):
    if s == 0:
        return x3
    zrow = jnp.zeros((HP, 1, DIM), jnp.float32)
    if s == -1:
        return jnp.concatenate([zrow, x3[:, :WP - 1]], axis=1)
    return jnp.concatenate([x3[:, 1:], zrow], axis=1)


def _row_shift(x3, s):
    if s == 0:
        return x3
    zpl = jnp.zeros((1, WP, DIM), jnp.float32)
    if s == -1:
        return jnp.concatenate([zpl, x3[:HP - 1]], axis=0)
    return jnp.concatenate([x3[1:], zpl], axis=0)


def _conv3x3_padded(x3, w_ref):
    cols = [_col_shift(x3, s) for s in (-1, 0, 1)]
    acc = None
    for t in range(9):
        dy, dx = t // 3, t % 3
        xs = _row_shift(cols[dx], dy - 1).reshape(HP * WP, DIM)
        z = jnp.dot(xs, w_ref[0, t], preferred_element_type=jnp.float32)
        acc = z if acc is None else acc + z
    return acc.reshape(HP, WP, DIM)


def _expert_kernel(ti_ref, gv_ref, xp_ref, w1_ref, b1_ref, w2_ref, b2_ref,
                   out_ref):
    b = pl.program_id(0)
    k = pl.program_id(1)

    ii = jax.lax.broadcasted_iota(jnp.int32, (HP, WP, 1), 0)
    jj = jax.lax.broadcasted_iota(jnp.int32, (HP, WP, 1), 1)
    interior = (ii >= 1) & (ii <= H) & (jj >= 1) & (jj <= W)

    x3 = xp_ref[0]
    h = _conv3x3_padded(x3, w1_ref) + b1_ref[0, 0][None, None, :]
    h = 0.5 * h * (1.0 + jax.lax.erf(h * 0.7071067811865476))
    h = jnp.where(interior, h, 0.0)

    y = _conv3x3_padded(h, w2_ref) + b2_ref[0, 0][None, None, :]
    contrib = y * gv_ref[b, k]

    @pl.when(k == 0)
    def _():
        out_ref[0] = contrib

    @pl.when(k != 0)
    def _():
        out_ref[0] = out_ref[0] + contrib


@jax.jit
def kernel(x, text_feature, Wg, bg, w1, b1, w2, b2):
    one = text_feature[0, 0] * 0.0 + 1.0
    xp = jnp.pad(x.transpose(0, 2, 3, 1),
                 ((0, 0), (1, 1), (1, 1), (0, 0))) * one
    w1r = w1.transpose(0, 3, 4, 2, 1).reshape(E, 9, DIM, DIM) * one
    w2r = w2.transpose(0, 3, 4, 2, 1).reshape(E, 9, DIM, DIM) * one
    b1r = b1.reshape(E, 1, DIM)
    b2r = b2.reshape(E, 1, DIM)

    ti, tv, aux = pl.pallas_call(
        _gate_kernel,
        out_shape=(
            jax.ShapeDtypeStruct((B, K), jnp.int32),
            jax.ShapeDtypeStruct((B, K), jnp.float32),
            jax.ShapeDtypeStruct((1, 1), jnp.float32),
        ),
    )(xp, text_feature, Wg[:DIM], Wg[DIM:], bg.reshape(1, E))

    grid_spec = pltpu.PrefetchScalarGridSpec(
        num_scalar_prefetch=2,
        grid=(B, K),
        in_specs=[
            pl.BlockSpec((1, HP, WP, DIM), lambda b, k, ti, tv: (b, 0, 0, 0)),
            pl.BlockSpec((1, 9, DIM, DIM),
                         lambda b, k, ti, tv: (ti[b, k], 0, 0, 0)),
            pl.BlockSpec((1, 1, DIM), lambda b, k, ti, tv: (ti[b, k], 0, 0)),
            pl.BlockSpec((1, 9, DIM, DIM),
                         lambda b, k, ti, tv: (ti[b, k], 0, 0, 0)),
            pl.BlockSpec((1, 1, DIM), lambda b, k, ti, tv: (ti[b, k], 0, 0)),
        ],
        out_specs=pl.BlockSpec((1, HP, WP, DIM),
                               lambda b, k, ti, tv: (b, 0, 0, 0)),
    )
    out = pl.pallas_call(
        _expert_kernel,
        grid_spec=grid_spec,
        out_shape=jax.ShapeDtypeStruct((B, HP, WP, DIM), jnp.float32),
    )(ti, tv, xp, w1r, b1r, w2r, b2r)

    return (out[:, 1:1 + H, 1:1 + W, :] * one).transpose(0, 3, 1, 2), aux[0, 0]


# speedup vs baseline: 2.4849x; 1.1833x over previous
import functools

import jax
import jax.numpy as jnp
from jax.experimental import pallas as pl
from jax.experimental.pallas import tpu as pltpu

B, DIM, H, W = 8, 192, 14, 14
D_TEXT = 512
E = 8
K = 2
HP, WP = H + 2, W + 2


def _gate_kernel(xp_ref, tf_ref, wgi_ref, wgt_ref, bg_ref,
                 ti_ref, tv_ref, aux_ref):
    xp = xp_ref[:]
    pooled = jnp.sum(xp, axis=(1, 2)) * (1.0 / (H * W))
    logits = (jnp.dot(pooled, wgi_ref[:], preferred_element_type=jnp.float32)
              + jnp.dot(tf_ref[:], wgt_ref[:], preferred_element_type=jnp.float32)
              + bg_ref[:])
    z = logits - jnp.max(logits, axis=-1, keepdims=True)
    ez = jnp.exp(z)
    probs = ez / jnp.sum(ez, axis=-1, keepdims=True)

    lane = jax.lax.broadcasted_iota(jnp.int32, (B, E), 1)
    v1 = jnp.max(probs, axis=-1, keepdims=True)
    i1 = jnp.min(jnp.where(probs == v1, lane, E), axis=-1, keepdims=True)
    p2 = jnp.where(lane == i1, -1.0, probs)
    v2 = jnp.max(p2, axis=-1, keepdims=True)
    i2 = jnp.min(jnp.where(p2 == v2, lane, E), axis=-1, keepdims=True)

    denom = 1.0 / (v1 + v2 + 1e-9)
    v1n = v1 * denom
    v2n = v2 * denom

    ti_ref[:] = jnp.concatenate([i1, i2], axis=1)
    tv_ref[:] = jnp.concatenate([v1n, v2n], axis=1)

    one1 = jnp.where(lane == i1, v1n, 0.0)
    one2 = jnp.where(lane == i2, v2n, 0.0)
    imp = jnp.sum(one1 + one2, axis=0, keepdims=True)
    m = jnp.mean(imp, axis=-1, keepdims=True)
    var = jnp.mean((imp - m) ** 2, axis=-1, keepdims=True)
    aux_ref[:] = var / (m * m + 1e-10)


def _col_shift(x3, s):
    if s == 0:
        return x3
    zrow = jnp.zeros((HP, 1, DIM), x3.dtype)
    if s == -1:
        return jnp.concatenate([zrow, x3[:, :WP - 1]], axis=1)
    return jnp.concatenate([x3[:, 1:], zrow], axis=1)


def _row_shift(x3, s):
    if s == 0:
        return x3
    zpl = jnp.zeros((1, WP, DIM), x3.dtype)
    if s == -1:
        return jnp.concatenate([zpl, x3[:HP - 1]], axis=0)
    return jnp.concatenate([x3[1:], zpl], axis=0)


def _conv3x3_padded(x3, w_ref):
    cols = [_col_shift(x3, s) for s in (-1, 0, 1)]
    acc = None
    for t in range(9):
        dy, dx = t // 3, t % 3
        xs = _row_shift(cols[dx], dy - 1).reshape(HP * WP, DIM)
        z = jnp.dot(xs, w_ref[0, t], preferred_element_type=jnp.float32)
        acc = z if acc is None else acc + z
    return acc.reshape(HP, WP, DIM)


def _expert_kernel(ti_ref, gv_ref, xp_ref, w1_ref, b1_ref, w2_ref, b2_ref,
                   out_ref):
    b = pl.program_id(0)
    k = pl.program_id(1)

    ii = jax.lax.broadcasted_iota(jnp.int32, (HP, WP, 1), 0)
    jj = jax.lax.broadcasted_iota(jnp.int32, (HP, WP, 1), 1)
    interior = (ii >= 1) & (ii <= H) & (jj >= 1) & (jj <= W)

    x3 = xp_ref[0]
    h = _conv3x3_padded(x3, w1_ref) + b1_ref[0, 0][None, None, :]
    h = 0.5 * h * (1.0 + jax.lax.erf(h * 0.7071067811865476))
    h = jnp.where(interior, h, 0.0).astype(jnp.bfloat16)

    y = _conv3x3_padded(h, w2_ref) + b2_ref[0, 0][None, None, :]
    contrib = y * gv_ref[b, k]

    @pl.when(k == 0)
    def _():
        out_ref[0] = contrib

    @pl.when(k != 0)
    def _():
        out_ref[0] = out_ref[0] + contrib


@jax.jit
def kernel(x, text_feature, Wg, bg, w1, b1, w2, b2):
    one = text_feature[0, 0] * 0.0 + 1.0
    xp = jnp.pad(x.transpose(0, 2, 3, 1),
                 ((0, 0), (1, 1), (1, 1), (0, 0))) * one
    xpb = xp.astype(jnp.bfloat16)
    w1r = (w1.transpose(0, 3, 4, 2, 1).reshape(E, 9, DIM, DIM)
           * one).astype(jnp.bfloat16)
    w2r = (w2.transpose(0, 3, 4, 2, 1).reshape(E, 9, DIM, DIM)
           * one).astype(jnp.bfloat16)
    b1r = b1.reshape(E, 1, DIM)
    b2r = b2.reshape(E, 1, DIM)

    ti, tv, aux = pl.pallas_call(
        _gate_kernel,
        out_shape=(
            jax.ShapeDtypeStruct((B, K), jnp.int32),
            jax.ShapeDtypeStruct((B, K), jnp.float32),
            jax.ShapeDtypeStruct((1, 1), jnp.float32),
        ),
    )(xp, text_feature, Wg[:DIM], Wg[DIM:], bg.reshape(1, E))

    grid_spec = pltpu.PrefetchScalarGridSpec(
        num_scalar_prefetch=2,
        grid=(B, K),
        in_specs=[
            pl.BlockSpec((1, HP, WP, DIM), lambda b, k, ti, tv: (b, 0, 0, 0)),
            pl.BlockSpec((1, 9, DIM, DIM),
                         lambda b, k, ti, tv: (ti[b, k], 0, 0, 0)),
            pl.BlockSpec((1, 1, DIM), lambda b, k, ti, tv: (ti[b, k], 0, 0)),
            pl.BlockSpec((1, 9, DIM, DIM),
                         lambda b, k, ti, tv: (ti[b, k], 0, 0, 0)),
            pl.BlockSpec((1, 1, DIM), lambda b, k, ti, tv: (ti[b, k], 0, 0)),
        ],
        out_specs=pl.BlockSpec((1, HP, WP, DIM),
                               lambda b, k, ti, tv: (b, 0, 0, 0)),
    )
    out = pl.pallas_call(
        _expert_kernel,
        grid_spec=grid_spec,
        out_shape=jax.ShapeDtypeStruct((B, HP, WP, DIM), jnp.float32),
    )(ti, tv, xpb, w1r, b1r, w2r, b2r)

    return (out[:, 1:1 + H, 1:1 + W, :] * one).transpose(0, 3, 1, 2), aux[0, 0]
